# baseline (speedup 1.0000x reference)
import functools

import jax
import jax.numpy as jnp
from jax import lax
from jax.experimental import pallas as pl
from jax.experimental.pallas import tpu as pltpu
from jax.experimental.pallas import tpu_sc as plsc

_H = 1024
_C = 32


def _make_sc_kernel(n_tokens):
    info = plsc.get_sparse_core_info()
    nw = info.num_cores * info.num_subcores
    tpw = n_tokens // nw
    nch = tpw // _C
    mesh = plsc.VectorSubcoreMesh(core_axis_name="c", subcore_axis_name="s")

    @functools.partial(
        pl.kernel,
        mesh=mesh,
        out_type=jax.ShapeDtypeStruct((n_tokens, _H), jnp.float32),
        scratch_types=[
            pltpu.VMEM((tpw,), jnp.int32),
            pltpu.VMEM((_C, _H), jnp.float32),
            pltpu.VMEM((_C, _H), jnp.float32),
            pltpu.SemaphoreType.DMA,
            pltpu.SemaphoreType.DMA,
            pltpu.SemaphoreType.DMA,
            pltpu.SemaphoreType.DMA,
        ],
    )
    def k(seg_hbm, table_hbm, out_hbm, idx_v, buf0, buf1, g0, g1, s0, s1):
        wid = lax.axis_index("s") * info.num_cores + lax.axis_index("c")
        base = wid * tpw
        pltpu.sync_copy(seg_hbm.at[pl.ds(base, tpw)], idx_v)

        bufs = (buf0, buf1)
        gsem = (g0, g1)
        ssem = (s0, s1)

        def gather(kk, b):
            return pltpu.async_copy(
                table_hbm.at[idx_v.at[pl.ds(kk * _C, _C)]], bufs[b], gsem[b])

        def scatter(kk, b):
            return pltpu.async_copy(
                bufs[b], out_hbm.at[pl.ds(base + kk * _C, _C)], ssem[b])

        pending_s = [None, None]
        g = gather(0, 0)
        for kk in range(nch):
            b = kk % 2
            g.wait()
            s = scatter(kk, b)
            if kk + 1 < nch:
                nb = 1 - b
                if pending_s[nb] is not None:
                    pending_s[nb].wait()
                g = gather(kk + 1, nb)
            pending_s[b] = s
        pending_s[0].wait()
        pending_s[1].wait()

    return k


def kernel(segments, table):
    b, s = segments.shape
    n = b * s
    out = _make_sc_kernel(n)(segments.reshape(n), table)
    return out.reshape(b, s, _H)


# speedup vs baseline: 4.4262x; 4.4262x over previous
import functools

import jax
import jax.numpy as jnp
from jax import lax
from jax.experimental import pallas as pl
from jax.experimental.pallas import tpu as pltpu
from jax.experimental.pallas import tpu_sc as plsc

_H = 1024
_G = 64


def _make_sc_kernel(n_tokens):
    info = plsc.get_sparse_core_info()
    nw = info.num_cores * info.num_subcores
    tpw = n_tokens // nw
    ng = tpw // _G
    mesh = plsc.VectorSubcoreMesh(core_axis_name="c", subcore_axis_name="s")

    @functools.partial(
        pl.kernel,
        mesh=mesh,
        out_type=jax.ShapeDtypeStruct((n_tokens * _H,), jnp.float32),
        scratch_types=[
            pltpu.VMEM_SHARED((info.num_subcores, tpw), jnp.int32),
            pltpu.SMEM((tpw,), jnp.int32),
            pltpu.VMEM((2 * _H,), jnp.float32),
            pltpu.SemaphoreType.DMA,
        ],
    )
    def k(seg_hbm, table_hbm, out_hbm, idx_sh, seg_s, table_v, sem):
        sid = lax.axis_index("s")
        wid = sid * info.num_cores + lax.axis_index("c")
        base = wid * tpw
        pltpu.sync_copy(seg_hbm.at[pl.ds(base, tpw)], idx_sh.at[sid])
        pltpu.sync_copy(idx_sh.at[sid], seg_s)
        pltpu.sync_copy(table_hbm, table_v)

        def drain_one_group():
            pltpu.make_async_copy(
                out_hbm.at[pl.ds(base * _H, _G * _H)],
                out_hbm.at[pl.ds(base * _H, _G * _H)],
                sem).wait()

        def tok(t, c):
            seg = seg_s[t]
            pltpu.make_async_copy(
                table_v.at[pl.ds(seg * _H, _H)],
                out_hbm.at[pl.ds((base + t) * _H, _H)],
                sem).start()
            return c

        def grp(g, c):
            lax.fori_loop(g * _G, (g + 1) * _G, tok, 0)

            @pl.when(g > 0)
            def _():
                drain_one_group()

            return c

        lax.fori_loop(0, ng, grp, 0)
        drain_one_group()

    return k


def kernel(segments, table):
    b, s = segments.shape
    n = b * s
    out = _make_sc_kernel(n)(segments.reshape(n), table.reshape(2 * _H))
    return out.reshape(b, s, _H)
